# speedup vs baseline: 105.2288x; 105.2288x over previous
import jax
import jax.numpy as jnp
from jax.experimental import pallas as pl
from jax.experimental.pallas import tpu as pltpu

B, C = 16384, 1000
BM = 512


def _loss_block(x, t, i):
    sp = jax.nn.sigmoid(x)
    sn = 1.0 - sp
    lp = jnp.log(jnp.clip(sp, 1e-8, None))
    ln = jnp.log(jnp.clip(sn, 1e-8, None))
    u = i == 0
    w_pos = jnp.where(u, sn * sn * sn, 1.0)
    w_neg = jnp.where(u, sp * sp * sp * sp, sp)
    val = jnp.where(t == 1, lp * w_pos, jnp.where(t == 0, ln * w_neg, 0.0))
    return jnp.sum(val)


def _kernel(logits_ref, targets_ref, idx_ref, out_ref, acc_ref):
    step = pl.program_id(0)

    @pl.when(step == 0)
    def _init():
        acc_ref[0] = 0.0

    acc_ref[0] += _loss_block(logits_ref[...], targets_ref[...], idx_ref[...])

    @pl.when(step == pl.num_programs(0) - 1)
    def _fin():
        out_ref[0] = -acc_ref[0]


def kernel(logits, targets, idx, prior_classes):
    del prior_classes
    grid = (B // BM,)
    in_spec = pl.BlockSpec((BM, C), lambda i: (i, 0))
    out = pl.pallas_call(
        _kernel,
        grid=grid,
        in_specs=[in_spec, in_spec, in_spec],
        out_specs=pl.BlockSpec(memory_space=pltpu.SMEM),
        out_shape=jax.ShapeDtypeStruct((1,), jnp.float32),
        scratch_shapes=[pltpu.SMEM((1,), jnp.float32)],
    )(logits, targets, idx)
    return out[0]


# speedup vs baseline: 116.1379x; 1.1037x over previous
import jax
import jax.numpy as jnp
from jax.experimental import pallas as pl
from jax.experimental.pallas import tpu as pltpu

B, C = 16384, 1000
BM = 512


def _loss_block(x, t, i):
    a = t == 1
    u = i == 0
    nm = t != -1
    q = jnp.where(a, x, -x)
    s = jax.nn.sigmoid(q)
    r = 1.0 - s
    l = jnp.log(jnp.maximum(s, 1e-8))
    r2 = r * r
    w34 = r2 * jnp.where(a, r, r2)
    w = jnp.where(u, w34, jnp.where(a, 1.0, r))
    val = jnp.where(nm, l * w, 0.0)
    return jnp.sum(val)


def _kernel(logits_ref, targets_ref, idx_ref, out_ref, acc_ref):
    step = pl.program_id(0)

    @pl.when(step == 0)
    def _init():
        acc_ref[0] = 0.0

    acc_ref[0] += _loss_block(logits_ref[...], targets_ref[...], idx_ref[...])

    @pl.when(step == pl.num_programs(0) - 1)
    def _fin():
        out_ref[0] = -acc_ref[0]


def kernel(logits, targets, idx, prior_classes):
    del prior_classes
    grid = (B // BM,)
    in_spec = pl.BlockSpec((BM, C), lambda i: (i, 0))
    out = pl.pallas_call(
        _kernel,
        grid=grid,
        in_specs=[in_spec, in_spec, in_spec],
        out_specs=pl.BlockSpec(memory_space=pltpu.SMEM),
        out_shape=jax.ShapeDtypeStruct((1,), jnp.float32),
        scratch_shapes=[pltpu.SMEM((1,), jnp.float32)],
    )(logits, targets, idx)
    return out[0]


# speedup vs baseline: 118.0303x; 1.0163x over previous
import jax
import jax.numpy as jnp
from jax.experimental import pallas as pl
from jax.experimental.pallas import tpu as pltpu

B, C = 16384, 1000
BM = 1024


def _loss_block(x, t, i):
    a = t == 1
    u = i == 0
    nm = t != -1
    q = jnp.where(a, x, -x)
    s = jax.nn.sigmoid(q)
    r = 1.0 - s
    l = jnp.log(jnp.maximum(s, 1e-8))
    r2 = r * r
    w34 = r2 * jnp.where(a, r, r2)
    w = jnp.where(u, w34, jnp.where(a, 1.0, r))
    val = jnp.where(nm, l * w, 0.0)
    return jnp.sum(val)


def _kernel(logits_ref, targets_ref, idx_ref, out_ref, acc_ref):
    step = pl.program_id(0)

    @pl.when(step == 0)
    def _init():
        acc_ref[0] = 0.0

    acc_ref[0] += _loss_block(logits_ref[...], targets_ref[...], idx_ref[...])

    @pl.when(step == pl.num_programs(0) - 1)
    def _fin():
        out_ref[0] = -acc_ref[0]


def kernel(logits, targets, idx, prior_classes):
    del prior_classes
    grid = (B // BM,)
    in_spec = pl.BlockSpec((BM, C), lambda i: (i, 0))
    out = pl.pallas_call(
        _kernel,
        grid=grid,
        in_specs=[in_spec, in_spec, in_spec],
        out_specs=pl.BlockSpec(memory_space=pltpu.SMEM),
        out_shape=jax.ShapeDtypeStruct((1,), jnp.float32),
        scratch_shapes=[pltpu.SMEM((1,), jnp.float32)],
    )(logits, targets, idx)
    return out[0]
